# baseline (speedup 1.0000x reference)
import functools

import jax
import jax.numpy as jnp
from jax import lax
from jax.experimental import pallas as pl
from jax.experimental.pallas import tpu as pltpu

B = 8
H = 224
GI = 168
GO = 84
BH = 56
GRID = (B, H // BH)


def _sel_mats():
    row = lax.broadcasted_iota(jnp.int32, (128, 128), 0)
    col = lax.broadcasted_iota(jnp.int32, (128, 128), 1)
    s0 = jnp.where((col < 64) & (row == 2 * col), 1.0, 0.0)
    s1 = jnp.where((col >= 64) & (row == 2 * (col - 64)), 1.0, 0.0)
    return s0.astype(jnp.float32), s1.astype(jnp.float32)


def _body(in_ref, out_ref):
    s0, s1 = _sel_mats()
    xe = in_ref[0, :, ::2, :].reshape(BH * GO, 128)
    xo = in_ref[0, :, 1::2, :].reshape(BH * GO, 128)
    y = jax.lax.dot(xe, s0, precision=lax.Precision.HIGHEST) + jax.lax.dot(
        xo, s1, precision=lax.Precision.HIGHEST
    )
    out_ref[...] = y.reshape(1, BH, GO, 128)


@jax.jit
def _tc_sel(xv):
    return pl.pallas_call(
        _body,
        grid=GRID,
        in_specs=[pl.BlockSpec((1, BH, GI, 128), lambda b, j: (b, j, 0, 0))],
        out_specs=pl.BlockSpec((1, BH, GO, 128), lambda b, j: (b, j, 0, 0)),
        out_shape=jax.ShapeDtypeStruct((B, H, GO, 128), jnp.float32),
    )(xv)


def kernel(inputs):
    xv = inputs.reshape(B, H, GI, 128)
    out = _tc_sel(xv)
    return out.reshape(8, 224, 224, 48)


# speedup vs baseline: 1.0524x; 1.0524x over previous
import functools

import jax
import jax.numpy as jnp
from jax import lax
from jax.experimental import pallas as pl
from jax.experimental.pallas import tpu as pltpu
from jax.experimental.pallas import tpu_sc as plsc

NR = 8 * 224
RW = 224 * 96
OW = RW // 2
NW = 32
RPW = NR // NW

_mesh = plsc.VectorSubcoreMesh(core_axis_name="c", subcore_axis_name="s")


@functools.partial(
    pl.kernel,
    mesh=_mesh,
    out_type=jax.ShapeDtypeStruct((NR, OW), jnp.float32),
    scratch_types=[
        pltpu.VMEM((1, RW), jnp.float32),
        pltpu.VMEM((1, RW), jnp.float32),
        pltpu.VMEM((1, OW), jnp.float32),
        pltpu.VMEM((1, OW), jnp.float32),
        pltpu.SemaphoreType.DMA,
        pltpu.SemaphoreType.DMA,
        pltpu.SemaphoreType.DMA,
        pltpu.SemaphoreType.DMA,
    ],
    compiler_params=pltpu.CompilerParams(needs_layout_passes=False),
)
def _sel(in_hbm, out_hbm, in0, in1, out0, out1, si0, si1, so0, so1):
    wid = lax.axis_index("s") * 2 + lax.axis_index("c")
    base = wid * RPW
    lanes2 = lax.iota(jnp.int32, 16) * 2
    zeros = jnp.zeros((16,), jnp.int32)

    def in_cp(i, buf, sem):
        row = pl.multiple_of(base + i, 1)
        return pltpu.make_async_copy(in_hbm.at[pl.ds(row, 1), :], buf, sem)

    def out_cp(i, buf, sem):
        row = pl.multiple_of(base + i, 1)
        return pltpu.make_async_copy(buf, out_hbm.at[pl.ds(row, 1), :], sem)

    def compute(src, dst):
        @plsc.parallel_loop(0, OW // 16, unroll=8)
        def _(k):
            dst[0, pl.ds(k * 16, 16)] = plsc.load_gather(
                src, [zeros, k * 32 + lanes2]
            )

    in_cp(0, in0, si0).start()

    def body(g, carry):
        i0 = g * 2
        i1 = i0 + 1

        in_cp(i0, in0, si0).wait()
        in_cp(i1, in1, si1).start()

        @pl.when(g > 0)
        def _():
            out_cp(i0 - 2, out0, so0).wait()

        compute(in0, out0)
        out_cp(i0, out0, so0).start()

        in_cp(i1, in1, si1).wait()

        @pl.when(g < RPW // 2 - 1)
        def _():
            in_cp(i0 + 2, in0, si0).start()

        @pl.when(g > 0)
        def _():
            out_cp(i1 - 2, out1, so1).wait()

        compute(in1, out1)
        out_cp(i1, out1, so1).start()
        return carry

    lax.fori_loop(0, RPW // 2, body, 0)
    out_cp(RPW - 2, out0, so0).wait()
    out_cp(RPW - 1, out1, so1).wait()


def kernel(inputs):
    mat = inputs.reshape(NR, RW)
    out = _sel(mat)
    return out.reshape(8, 224, 224, 48)


# speedup vs baseline: 1.6568x; 1.5742x over previous
import functools

import jax
import jax.numpy as jnp
from jax import lax
from jax.experimental import pallas as pl
from jax.experimental.pallas import tpu as pltpu
from jax.experimental.pallas import tpu_sc as plsc

N = 8 * 224 * 224
C = 96
OC = 48
NW = 32
RPW = N // NW
CR = 224
NITER = RPW // CR

_mesh = plsc.VectorSubcoreMesh(core_axis_name="c", subcore_axis_name="s")


@functools.partial(
    pl.kernel,
    mesh=_mesh,
    out_type=jax.ShapeDtypeStruct((N, OC), jnp.float32),
    scratch_types=[
        pltpu.VMEM((CR, C), jnp.float32),
        pltpu.VMEM((CR, C), jnp.float32),
        pltpu.VMEM((CR, OC), jnp.float32),
        pltpu.VMEM((CR, OC), jnp.float32),
        pltpu.SemaphoreType.DMA,
        pltpu.SemaphoreType.DMA,
        pltpu.SemaphoreType.DMA,
        pltpu.SemaphoreType.DMA,
    ],
    compiler_params=pltpu.CompilerParams(needs_layout_passes=False),
)
def _sel(in_hbm, out_hbm, in0, in1, out0, out1, si0, si1, so0, so1):
    wid = lax.axis_index("s") * 2 + lax.axis_index("c")
    base = wid * RPW
    lanes = lax.iota(jnp.int32, 16)
    lanes2 = lanes * 2

    def in_cp(i, buf, sem):
        row = pl.multiple_of(base + i * CR, 8)
        return pltpu.make_async_copy(in_hbm.at[pl.ds(row, CR), :], buf, sem)

    def out_cp(i, buf, sem):
        row = pl.multiple_of(base + i * CR, 8)
        return pltpu.make_async_copy(buf, out_hbm.at[pl.ds(row, CR), :], sem)

    def compute(src, dst):
        @plsc.parallel_loop(0, CR, unroll=4)
        def _(r):
            rows = jnp.full((16,), r, jnp.int32)
            for k in range(OC // 16):
                dst[r, pl.ds(k * 16, 16)] = plsc.load_gather(
                    src, [rows, k * 32 + lanes2]
                )

    in_cp(0, in0, si0).start()

    def body(g, carry):
        i0 = g * 2
        i1 = i0 + 1

        in_cp(i0, in0, si0).wait()
        in_cp(i1, in1, si1).start()

        @pl.when(g > 0)
        def _():
            out_cp(i0 - 2, out0, so0).wait()

        compute(in0, out0)
        out_cp(i0, out0, so0).start()

        in_cp(i1, in1, si1).wait()

        @pl.when(g < NITER // 2 - 1)
        def _():
            in_cp(i0 + 2, in0, si0).start()

        @pl.when(g > 0)
        def _():
            out_cp(i1 - 2, out1, so1).wait()

        compute(in1, out1)
        out_cp(i1, out1, so1).start()
        return carry

    lax.fori_loop(0, NITER // 2, body, 0)
    out_cp(NITER - 2, out0, so0).wait()
    out_cp(NITER - 1, out1, so1).wait()


def kernel(inputs):
    mat = inputs.reshape(N, C)
    out = _sel(mat)
    return out.reshape(8, 224, 224, 48)
